# baseline (device time: 234419 ns/iter reference)
import jax
import jax.numpy as jnp
from jax import lax
from jax.experimental import pallas as pl
from jax.experimental.pallas import tpu as pltpu

N_DEV = 32
SCALE = 0.08838834764831843


def _ring_allreduce(partial):
    n, rows, cols = partial.shape

    def body(p_ref, o_ref, comm_ref, rs_send, rs_recv, ag_send, ag_recv):
        my = lax.axis_index("i")
        left = (my + N_DEV - 1) % N_DEV
        right = (my + 1) % N_DEV

        barrier_sem = pltpu.get_barrier_semaphore()
        for nbr in (left, right):
            pl.semaphore_signal(
                barrier_sem, inc=1,
                device_id=(nbr,), device_id_type=pl.DeviceIdType.MESH,
            )
        pl.semaphore_wait(barrier_sem, 2)

        o_ref[...] = p_ref[...]

        for s in range(N_DEV - 1):
            sc = (my + 2 * N_DEV - s) % N_DEV
            rc = (my + 2 * N_DEV - s - 1) % N_DEV
            rdma = pltpu.make_async_remote_copy(
                src_ref=o_ref.at[sc],
                dst_ref=comm_ref.at[s],
                send_sem=rs_send.at[s],
                recv_sem=rs_recv.at[s],
                device_id=(right,),
                device_id_type=pl.DeviceIdType.MESH,
            )
            rdma.start()
            rdma.wait()
            o_ref[rc] = o_ref[rc] + comm_ref[s]

        for s in range(N_DEV - 1):
            sc = (my + 2 * N_DEV + 1 - s) % N_DEV
            rdma = pltpu.make_async_remote_copy(
                src_ref=o_ref.at[sc],
                dst_ref=o_ref.at[sc],
                send_sem=ag_send.at[s],
                recv_sem=ag_recv.at[s],
                device_id=(right,),
                device_id_type=pl.DeviceIdType.MESH,
            )
            rdma.start()
            rdma.wait()

    return pl.pallas_call(
        body,
        out_shape=jax.ShapeDtypeStruct((n, rows, cols), jnp.float32),
        in_specs=[pl.BlockSpec(memory_space=pltpu.VMEM)],
        out_specs=pl.BlockSpec(memory_space=pltpu.VMEM),
        scratch_shapes=[
            pltpu.VMEM((N_DEV - 1, rows, cols), jnp.float32),
            pltpu.SemaphoreType.DMA((N_DEV - 1,)),
            pltpu.SemaphoreType.DMA((N_DEV - 1,)),
            pltpu.SemaphoreType.DMA((N_DEV - 1,)),
            pltpu.SemaphoreType.DMA((N_DEV - 1,)),
        ],
        compiler_params=pltpu.CompilerParams(collective_id=0),
    )(partial)


def kernel(x, Wq, Wo, Wk, Wv):
    i = lax.axis_index("i")
    B, Sq, D = x.shape
    Hq_loc, Dh = 8, 128
    bf = jnp.bfloat16

    x2 = x.reshape(B * Sq, D).astype(bf)
    Q = (x2 @ Wq.astype(bf)).reshape(B, Sq, Hq_loc, Dh)

    Wk_s = lax.dynamic_slice_in_dim(Wk, i * 256, 256, axis=1).astype(bf)
    Wv_s = lax.dynamic_slice_in_dim(Wv, i * 256, 256, axis=1).astype(bf)
    K = (x2 @ Wk_s).reshape(B, Sq, 2, Dh)
    V = (x2 @ Wv_s).reshape(B, Sq, 2, Dh)
    Kr = jnp.repeat(K, 4, axis=2)
    Vr = jnp.repeat(V, 4, axis=2)

    s = jnp.einsum(
        "bihd,bjhd->bhij", Q, Kr, preferred_element_type=jnp.float32
    ) * SCALE
    p = jax.nn.softmax(s, axis=-1)
    o = jnp.einsum(
        "bhij,bjhd->bihd", p.astype(bf), Vr, preferred_element_type=jnp.float32
    )

    partial = jnp.dot(
        o.reshape(B * Sq, Hq_loc * Dh).astype(bf),
        Wo.astype(bf),
        preferred_element_type=jnp.float32,
    )

    red = _ring_allreduce(partial.reshape(N_DEV, (B * Sq) // N_DEV, D))
    return red.reshape(B, Sq, D)


# device time: 190891 ns/iter; 1.2280x vs baseline; 1.2280x over previous
import jax
import jax.numpy as jnp
from jax import lax
from jax.experimental import pallas as pl
from jax.experimental.pallas import tpu as pltpu

N_DEV = 32
SCALE = 0.08838834764831843


def _ring_allreduce(partial):
    n, rows, cols = partial.shape
    dt = partial.dtype

    def body(p_ref, o_ref, comm_ref, rs_send, rs_recv, ag_send, ag_recv):
        my = lax.axis_index("i")
        left = (my + N_DEV - 1) % N_DEV
        right = (my + 1) % N_DEV

        barrier_sem = pltpu.get_barrier_semaphore()
        for nbr in (left, right):
            pl.semaphore_signal(
                barrier_sem, inc=1,
                device_id=(nbr,), device_id_type=pl.DeviceIdType.MESH,
            )
        pl.semaphore_wait(barrier_sem, 2)

        o_ref[...] = p_ref[...]

        for s in range(N_DEV - 1):
            sc = (my + 2 * N_DEV - s) % N_DEV
            rc = (my + 2 * N_DEV - s - 1) % N_DEV
            rdma = pltpu.make_async_remote_copy(
                src_ref=o_ref.at[sc],
                dst_ref=comm_ref.at[s],
                send_sem=rs_send.at[s],
                recv_sem=rs_recv.at[s],
                device_id=(right,),
                device_id_type=pl.DeviceIdType.MESH,
            )
            rdma.start()
            rdma.wait()
            o_ref[rc] = o_ref[rc] + comm_ref[s]

        for s in range(N_DEV - 1):
            sc = (my + 2 * N_DEV + 1 - s) % N_DEV
            rdma = pltpu.make_async_remote_copy(
                src_ref=o_ref.at[sc],
                dst_ref=o_ref.at[sc],
                send_sem=ag_send.at[s],
                recv_sem=ag_recv.at[s],
                device_id=(right,),
                device_id_type=pl.DeviceIdType.MESH,
            )
            rdma.start()
            rdma.wait()

    return pl.pallas_call(
        body,
        out_shape=jax.ShapeDtypeStruct((n, rows, cols), dt),
        in_specs=[pl.BlockSpec(memory_space=pltpu.VMEM)],
        out_specs=pl.BlockSpec(memory_space=pltpu.VMEM),
        scratch_shapes=[
            pltpu.VMEM((N_DEV - 1, rows, cols), dt),
            pltpu.SemaphoreType.DMA((N_DEV - 1,)),
            pltpu.SemaphoreType.DMA((N_DEV - 1,)),
            pltpu.SemaphoreType.DMA((N_DEV - 1,)),
            pltpu.SemaphoreType.DMA((N_DEV - 1,)),
        ],
        compiler_params=pltpu.CompilerParams(collective_id=0),
    )(partial)


def kernel(x, Wq, Wo, Wk, Wv):
    i = lax.axis_index("i")
    B, Sq, D = x.shape
    Hq_loc, Dh = 8, 128
    bf = jnp.bfloat16

    x2 = x.reshape(B * Sq, D).astype(bf)
    Q = (x2 @ Wq.astype(bf)).reshape(B, Sq, Hq_loc, Dh)

    Wk_s = lax.dynamic_slice_in_dim(Wk, i * 256, 256, axis=1).astype(bf)
    Wv_s = lax.dynamic_slice_in_dim(Wv, i * 256, 256, axis=1).astype(bf)
    K = (x2 @ Wk_s).reshape(B, Sq, 2, Dh)
    V = (x2 @ Wv_s).reshape(B, Sq, 2, Dh)
    Kr = jnp.repeat(K, 4, axis=2)
    Vr = jnp.repeat(V, 4, axis=2)

    s = jnp.einsum(
        "bihd,bjhd->bhij", Q, Kr, preferred_element_type=jnp.float32
    ) * SCALE
    p = jax.nn.softmax(s, axis=-1)
    o = jnp.einsum(
        "bhij,bjhd->bihd", p.astype(bf), Vr, preferred_element_type=jnp.float32
    )

    partial = jnp.dot(
        o.reshape(B * Sq, Hq_loc * Dh).astype(bf),
        Wo.astype(bf),
        preferred_element_type=jnp.float32,
    )

    red = _ring_allreduce(
        partial.reshape(N_DEV, (B * Sq) // N_DEV, D).astype(bf)
    )
    return red.astype(jnp.float32).reshape(B, Sq, D)


# device time: 113583 ns/iter; 2.0639x vs baseline; 1.6806x over previous
import jax
import jax.numpy as jnp
from jax import lax
from jax.experimental import pallas as pl
from jax.experimental.pallas import tpu as pltpu

N_DEV = 32
SCALE = 0.08838834764831843


def _allreduce_2d(partial):
    n, rows, cols = partial.shape
    dt = partial.dtype

    def body(p_ref, o_ref, comm_a, a_acc, comm_b,
             a_send, a_recv, brs_send, brs_recv, bag_send, bag_recv,
             c_send, c_recv):
        my = lax.axis_index("i")
        g = my % 8
        z = my // 8
        z_next = g + 8 * ((z + 1) % 4)
        z_prev = g + 8 * ((z + 3) % 4)
        p_next = 8 * z + (g + 1) % 8
        p_prev = 8 * z + (g + 7) % 8

        barrier_sem = pltpu.get_barrier_semaphore()
        for nbr in (z_next, z_prev, p_next, p_prev):
            pl.semaphore_signal(
                barrier_sem, inc=1,
                device_id=(nbr,), device_id_type=pl.DeviceIdType.MESH,
            )
        pl.semaphore_wait(barrier_sem, 4)

        for s in range(3):
            st = (z + 8 - s) % 4
            rt = (z + 8 - s - 1) % 4
            src = p_ref.at[pl.ds(st * 8, 8)] if s == 0 else a_acc.at[s - 1]
            rdma = pltpu.make_async_remote_copy(
                src_ref=src,
                dst_ref=comm_a.at[s],
                send_sem=a_send.at[s],
                recv_sem=a_recv.at[s],
                device_id=(z_next,),
                device_id_type=pl.DeviceIdType.MESH,
            )
            rdma.start()
            rdma.wait()
            a_acc[s] = comm_a[s] + p_ref[pl.ds(rt * 8, 8)]

        t = (z + 1) % 4
        base = t * 8
        o_ref[pl.ds(base, 8)] = a_acc[2]

        for s in range(7):
            cs = base + (g + 16 - s) % 8
            cr = base + (g + 16 - s - 1) % 8
            rdma = pltpu.make_async_remote_copy(
                src_ref=o_ref.at[cs],
                dst_ref=comm_b.at[s],
                send_sem=brs_send.at[s],
                recv_sem=brs_recv.at[s],
                device_id=(p_next,),
                device_id_type=pl.DeviceIdType.MESH,
            )
            rdma.start()
            rdma.wait()
            o_ref[cr] = o_ref[cr] + comm_b[s]

        for s in range(7):
            cs = base + (g + 17 - s) % 8
            rdma = pltpu.make_async_remote_copy(
                src_ref=o_ref.at[cs],
                dst_ref=o_ref.at[cs],
                send_sem=bag_send.at[s],
                recv_sem=bag_recv.at[s],
                device_id=(p_next,),
                device_id_type=pl.DeviceIdType.MESH,
            )
            rdma.start()
            rdma.wait()

        for s in range(3):
            st = (z + 9 - s) % 4
            rdma = pltpu.make_async_remote_copy(
                src_ref=o_ref.at[pl.ds(st * 8, 8)],
                dst_ref=o_ref.at[pl.ds(st * 8, 8)],
                send_sem=c_send.at[s],
                recv_sem=c_recv.at[s],
                device_id=(z_next,),
                device_id_type=pl.DeviceIdType.MESH,
            )
            rdma.start()
            rdma.wait()

    return pl.pallas_call(
        body,
        out_shape=jax.ShapeDtypeStruct((n, rows, cols), dt),
        in_specs=[pl.BlockSpec(memory_space=pltpu.VMEM)],
        out_specs=pl.BlockSpec(memory_space=pltpu.VMEM),
        scratch_shapes=[
            pltpu.VMEM((3, 8, rows, cols), dt),
            pltpu.VMEM((3, 8, rows, cols), dt),
            pltpu.VMEM((7, rows, cols), dt),
            pltpu.SemaphoreType.DMA((3,)),
            pltpu.SemaphoreType.DMA((3,)),
            pltpu.SemaphoreType.DMA((7,)),
            pltpu.SemaphoreType.DMA((7,)),
            pltpu.SemaphoreType.DMA((7,)),
            pltpu.SemaphoreType.DMA((7,)),
            pltpu.SemaphoreType.DMA((3,)),
            pltpu.SemaphoreType.DMA((3,)),
        ],
        compiler_params=pltpu.CompilerParams(collective_id=0),
    )(partial)


def _ring_allreduce(partial):
    n, rows, cols = partial.shape
    dt = partial.dtype

    def body(p_ref, o_ref, comm_ref, rs_send, rs_recv, ag_send, ag_recv):
        my = lax.axis_index("i")
        left = (my + N_DEV - 1) % N_DEV
        right = (my + 1) % N_DEV

        barrier_sem = pltpu.get_barrier_semaphore()
        for nbr in (left, right):
            pl.semaphore_signal(
                barrier_sem, inc=1,
                device_id=(nbr,), device_id_type=pl.DeviceIdType.MESH,
            )
        pl.semaphore_wait(barrier_sem, 2)

        o_ref[...] = p_ref[...]

        for s in range(N_DEV - 1):
            sc = (my + 2 * N_DEV - s) % N_DEV
            rc = (my + 2 * N_DEV - s - 1) % N_DEV
            rdma = pltpu.make_async_remote_copy(
                src_ref=o_ref.at[sc],
                dst_ref=comm_ref.at[s],
                send_sem=rs_send.at[s],
                recv_sem=rs_recv.at[s],
                device_id=(right,),
                device_id_type=pl.DeviceIdType.MESH,
            )
            rdma.start()
            rdma.wait()
            o_ref[rc] = o_ref[rc] + comm_ref[s]

        for s in range(N_DEV - 1):
            sc = (my + 2 * N_DEV + 1 - s) % N_DEV
            rdma = pltpu.make_async_remote_copy(
                src_ref=o_ref.at[sc],
                dst_ref=o_ref.at[sc],
                send_sem=ag_send.at[s],
                recv_sem=ag_recv.at[s],
                device_id=(right,),
                device_id_type=pl.DeviceIdType.MESH,
            )
            rdma.start()
            rdma.wait()

    return pl.pallas_call(
        body,
        out_shape=jax.ShapeDtypeStruct((n, rows, cols), dt),
        in_specs=[pl.BlockSpec(memory_space=pltpu.VMEM)],
        out_specs=pl.BlockSpec(memory_space=pltpu.VMEM),
        scratch_shapes=[
            pltpu.VMEM((N_DEV - 1, rows, cols), dt),
            pltpu.SemaphoreType.DMA((N_DEV - 1,)),
            pltpu.SemaphoreType.DMA((N_DEV - 1,)),
            pltpu.SemaphoreType.DMA((N_DEV - 1,)),
            pltpu.SemaphoreType.DMA((N_DEV - 1,)),
        ],
        compiler_params=pltpu.CompilerParams(collective_id=0),
    )(partial)


def kernel(x, Wq, Wo, Wk, Wv):
    i = lax.axis_index("i")
    B, Sq, D = x.shape
    Hq_loc, Dh = 8, 128
    bf = jnp.bfloat16

    x2 = x.reshape(B * Sq, D).astype(bf)
    Q = (x2 @ Wq.astype(bf)).reshape(B, Sq, Hq_loc, Dh)

    Wk_s = lax.dynamic_slice_in_dim(Wk, i * 256, 256, axis=1).astype(bf)
    Wv_s = lax.dynamic_slice_in_dim(Wv, i * 256, 256, axis=1).astype(bf)
    K = (x2 @ Wk_s).reshape(B, Sq, 2, Dh)
    V = (x2 @ Wv_s).reshape(B, Sq, 2, Dh)
    Kr = jnp.repeat(K, 4, axis=2)
    Vr = jnp.repeat(V, 4, axis=2)

    s = jnp.einsum(
        "bihd,bjhd->bhij", Q, Kr, preferred_element_type=jnp.float32
    ) * SCALE
    p = jax.nn.softmax(s, axis=-1)
    o = jnp.einsum(
        "bhij,bjhd->bihd", p.astype(bf), Vr, preferred_element_type=jnp.float32
    )

    partial = jnp.dot(
        o.reshape(B * Sq, Hq_loc * Dh).astype(bf),
        Wo.astype(bf),
        preferred_element_type=jnp.float32,
    )

    red = _allreduce_2d(
        partial.reshape(N_DEV, (B * Sq) // N_DEV, D).astype(bf)
    )
    return red.astype(jnp.float32).reshape(B, Sq, D)


# device time: 98224 ns/iter; 2.3866x vs baseline; 1.1564x over previous
import jax
import jax.numpy as jnp
from jax import lax
from jax.experimental import pallas as pl
from jax.experimental.pallas import tpu as pltpu

N_DEV = 32
SCALE = 0.08838834764831843


def _allreduce_2d(partial):
    n, rows, cols = partial.shape
    dt = partial.dtype

    def body(p_ref, o_ref, comm_a, a_acc, comm_b,
             a_send, a_recv, brs_send, brs_recv, bag_send, bag_recv,
             c_send, c_recv):
        my = lax.axis_index("i")
        g = my % 8
        z = my // 8
        z_next = g + 8 * ((z + 1) % 4)
        z_prev = g + 8 * ((z + 3) % 4)

        barrier_sem = pltpu.get_barrier_semaphore()
        for nbr in (z_next, z_prev, 8 * z + (g ^ 1), 8 * z + (g ^ 2),
                    8 * z + (g ^ 4)):
            pl.semaphore_signal(
                barrier_sem, inc=1,
                device_id=(nbr,), device_id_type=pl.DeviceIdType.MESH,
            )
        pl.semaphore_wait(barrier_sem, 5)

        for s in range(3):
            st = (z + 8 - s) % 4
            rt = (z + 8 - s - 1) % 4
            src = p_ref.at[pl.ds(st * 8, 8)] if s == 0 else a_acc.at[s - 1]
            rdma = pltpu.make_async_remote_copy(
                src_ref=src,
                dst_ref=comm_a.at[s],
                send_sem=a_send.at[s],
                recv_sem=a_recv.at[s],
                device_id=(z_next,),
                device_id_type=pl.DeviceIdType.MESH,
            )
            rdma.start()
            rdma.wait()
            a_acc[s] = comm_a[s] + p_ref[pl.ds(rt * 8, 8)]

        t = (z + 1) % 4
        base = t * 8
        o_ref[pl.ds(base, 8)] = a_acc[2]

        rs_off = (0, 4, 6)
        for k in range(3):
            m = 1 << k
            width = 2 * m
            partner = 8 * z + (g ^ m)
            s_low = (g ^ m) % width
            r_low = g % width
            rdmas = []
            for j in range(8 // width):
                slot = rs_off[k] + j
                rdma = pltpu.make_async_remote_copy(
                    src_ref=o_ref.at[base + s_low + j * width],
                    dst_ref=comm_b.at[slot],
                    send_sem=brs_send.at[slot],
                    recv_sem=brs_recv.at[slot],
                    device_id=(partner,),
                    device_id_type=pl.DeviceIdType.MESH,
                )
                rdma.start()
                rdmas.append(rdma)
            for j, rdma in enumerate(rdmas):
                rdma.wait()
                rc = base + r_low + j * width
                o_ref[rc] = o_ref[rc] + comm_b[rs_off[k] + j]

        ag_off = (0, 1, 3)
        for k in range(3):
            m = 4 >> k
            span = 8 >> k
            partner = 8 * z + (g ^ m)
            own_low = g % span
            rdmas = []
            for j in range(1 << k):
                slot = ag_off[k] + j
                rdma = pltpu.make_async_remote_copy(
                    src_ref=o_ref.at[base + own_low + j * span],
                    dst_ref=o_ref.at[base + own_low + j * span],
                    send_sem=bag_send.at[slot],
                    recv_sem=bag_recv.at[slot],
                    device_id=(partner,),
                    device_id_type=pl.DeviceIdType.MESH,
                )
                rdma.start()
                rdmas.append(rdma)
            for rdma in rdmas:
                rdma.wait()

        for s in range(3):
            st = (z + 9 - s) % 4
            rdma = pltpu.make_async_remote_copy(
                src_ref=o_ref.at[pl.ds(st * 8, 8)],
                dst_ref=o_ref.at[pl.ds(st * 8, 8)],
                send_sem=c_send.at[s],
                recv_sem=c_recv.at[s],
                device_id=(z_next,),
                device_id_type=pl.DeviceIdType.MESH,
            )
            rdma.start()
            rdma.wait()

    return pl.pallas_call(
        body,
        out_shape=jax.ShapeDtypeStruct((n, rows, cols), dt),
        in_specs=[pl.BlockSpec(memory_space=pltpu.VMEM)],
        out_specs=pl.BlockSpec(memory_space=pltpu.VMEM),
        scratch_shapes=[
            pltpu.VMEM((3, 8, rows, cols), dt),
            pltpu.VMEM((3, 8, rows, cols), dt),
            pltpu.VMEM((7, rows, cols), dt),
            pltpu.SemaphoreType.DMA((3,)),
            pltpu.SemaphoreType.DMA((3,)),
            pltpu.SemaphoreType.DMA((7,)),
            pltpu.SemaphoreType.DMA((7,)),
            pltpu.SemaphoreType.DMA((7,)),
            pltpu.SemaphoreType.DMA((7,)),
            pltpu.SemaphoreType.DMA((3,)),
            pltpu.SemaphoreType.DMA((3,)),
        ],
        compiler_params=pltpu.CompilerParams(collective_id=0),
    )(partial)


def _ring_allreduce(partial):
    n, rows, cols = partial.shape
    dt = partial.dtype

    def body(p_ref, o_ref, comm_ref, rs_send, rs_recv, ag_send, ag_recv):
        my = lax.axis_index("i")
        left = (my + N_DEV - 1) % N_DEV
        right = (my + 1) % N_DEV

        barrier_sem = pltpu.get_barrier_semaphore()
        for nbr in (left, right):
            pl.semaphore_signal(
                barrier_sem, inc=1,
                device_id=(nbr,), device_id_type=pl.DeviceIdType.MESH,
            )
        pl.semaphore_wait(barrier_sem, 2)

        o_ref[...] = p_ref[...]

        for s in range(N_DEV - 1):
            sc = (my + 2 * N_DEV - s) % N_DEV
            rc = (my + 2 * N_DEV - s - 1) % N_DEV
            rdma = pltpu.make_async_remote_copy(
                src_ref=o_ref.at[sc],
                dst_ref=comm_ref.at[s],
                send_sem=rs_send.at[s],
                recv_sem=rs_recv.at[s],
                device_id=(right,),
                device_id_type=pl.DeviceIdType.MESH,
            )
            rdma.start()
            rdma.wait()
            o_ref[rc] = o_ref[rc] + comm_ref[s]

        for s in range(N_DEV - 1):
            sc = (my + 2 * N_DEV + 1 - s) % N_DEV
            rdma = pltpu.make_async_remote_copy(
                src_ref=o_ref.at[sc],
                dst_ref=o_ref.at[sc],
                send_sem=ag_send.at[s],
                recv_sem=ag_recv.at[s],
                device_id=(right,),
                device_id_type=pl.DeviceIdType.MESH,
            )
            rdma.start()
            rdma.wait()

    return pl.pallas_call(
        body,
        out_shape=jax.ShapeDtypeStruct((n, rows, cols), dt),
        in_specs=[pl.BlockSpec(memory_space=pltpu.VMEM)],
        out_specs=pl.BlockSpec(memory_space=pltpu.VMEM),
        scratch_shapes=[
            pltpu.VMEM((N_DEV - 1, rows, cols), dt),
            pltpu.SemaphoreType.DMA((N_DEV - 1,)),
            pltpu.SemaphoreType.DMA((N_DEV - 1,)),
            pltpu.SemaphoreType.DMA((N_DEV - 1,)),
            pltpu.SemaphoreType.DMA((N_DEV - 1,)),
        ],
        compiler_params=pltpu.CompilerParams(collective_id=0),
    )(partial)


def kernel(x, Wq, Wo, Wk, Wv):
    i = lax.axis_index("i")
    B, Sq, D = x.shape
    Hq_loc, Dh = 8, 128
    bf = jnp.bfloat16

    x2 = x.reshape(B * Sq, D).astype(bf)
    Q = (x2 @ Wq.astype(bf)).reshape(B, Sq, Hq_loc, Dh)

    Wk_s = lax.dynamic_slice_in_dim(Wk, i * 256, 256, axis=1).astype(bf)
    Wv_s = lax.dynamic_slice_in_dim(Wv, i * 256, 256, axis=1).astype(bf)
    K = (x2 @ Wk_s).reshape(B, Sq, 2, Dh)
    V = (x2 @ Wv_s).reshape(B, Sq, 2, Dh)
    Kr = jnp.repeat(K, 4, axis=2)
    Vr = jnp.repeat(V, 4, axis=2)

    s = jnp.einsum(
        "bihd,bjhd->bhij", Q, Kr, preferred_element_type=jnp.float32
    ) * SCALE
    p = jax.nn.softmax(s, axis=-1)
    o = jnp.einsum(
        "bhij,bjhd->bihd", p.astype(bf), Vr, preferred_element_type=jnp.float32
    )

    partial = jnp.dot(
        o.reshape(B * Sq, Hq_loc * Dh).astype(bf),
        Wo.astype(bf),
        preferred_element_type=jnp.float32,
    )

    red = _allreduce_2d(
        partial.reshape(N_DEV, (B * Sq) // N_DEV, D).astype(bf)
    )
    return red.astype(jnp.float32).reshape(B, Sq, D)


# device time: 91750 ns/iter; 2.5550x vs baseline; 1.0706x over previous
import jax
import jax.numpy as jnp
from jax import lax
from jax.experimental import pallas as pl
from jax.experimental.pallas import tpu as pltpu

N_DEV = 32
SCALE = 0.08838834764831843


def _allreduce_2d(partial):
    n, rows, cols = partial.shape
    dt = partial.dtype

    def body(p_ref, o_ref, comm_a, a_acc, comm_b,
             a_send, a_recv, brs_send, brs_recv, bag_send, bag_recv,
             c_send, c_recv):
        my = lax.axis_index("i")
        g = my % 8
        z = my // 8
        z_next = g + 8 * ((z + 1) % 4)
        z_prev = g + 8 * ((z + 3) % 4)

        barrier_sem = pltpu.get_barrier_semaphore()
        for nbr in (z_next, z_prev, 8 * z + (g ^ 1), 8 * z + (g ^ 2),
                    8 * z + (g ^ 4)):
            pl.semaphore_signal(
                barrier_sem, inc=1,
                device_id=(nbr,), device_id_type=pl.DeviceIdType.MESH,
            )
        pl.semaphore_wait(barrier_sem, 5)

        for s in range(3):
            st = (z + 8 - s) % 4
            rt = (z + 8 - s - 1) % 4
            src = p_ref.at[pl.ds(st * 8, 8)] if s == 0 else a_acc.at[s - 1]
            rdma = pltpu.make_async_remote_copy(
                src_ref=src,
                dst_ref=comm_a.at[s],
                send_sem=a_send.at[s],
                recv_sem=a_recv.at[s],
                device_id=(z_next,),
                device_id_type=pl.DeviceIdType.MESH,
            )
            rdma.start()
            rdma.wait()
            a_acc[s] = comm_a[s] + p_ref[pl.ds(rt * 8, 8)]

        t = (z + 1) % 4
        base = t * 8
        o_ref[pl.ds(base, 8)] = a_acc[2]

        rs_off = (0, 4, 6)
        for k in range(3):
            m = 1 << k
            width = 2 * m
            partner = 8 * z + (g ^ m)
            s_low = (g ^ m) % width
            r_low = g % width
            rdmas = []
            for j in range(8 // width):
                slot = rs_off[k] + j
                rdma = pltpu.make_async_remote_copy(
                    src_ref=o_ref.at[base + s_low + j * width],
                    dst_ref=comm_b.at[slot],
                    send_sem=brs_send.at[slot],
                    recv_sem=brs_recv.at[slot],
                    device_id=(partner,),
                    device_id_type=pl.DeviceIdType.MESH,
                )
                rdma.start()
                rdmas.append(rdma)
            for j, rdma in enumerate(rdmas):
                rdma.wait()
                rc = base + r_low + j * width
                o_ref[rc] = o_ref[rc] + comm_b[rs_off[k] + j]

        def slot_chunk(j):
            if j == 0:
                return g
            if j == 1:
                return g ^ 4
            if j < 4:
                return (g ^ 2) % 4 + 4 * (j - 2)
            return (g ^ 1) % 2 + 2 * (j - 4)

        def c_desc(h, j):
            idx = ((z + 9 - h) % 4) * 8 + slot_chunk(j)
            return pltpu.make_async_remote_copy(
                src_ref=o_ref.at[idx],
                dst_ref=o_ref.at[idx],
                send_sem=c_send.at[8 * h + j],
                recv_sem=c_recv.at[8 * h + j],
                device_id=(z_next,),
                device_id_type=pl.DeviceIdType.MESH,
            )

        c_desc(0, 0).start()

        ag_off = (0, 1, 3)
        for k in range(3):
            m = 4 >> k
            span = 8 >> k
            partner = 8 * z + (g ^ m)
            own_low = g % span
            rdmas = []
            for j in range(1 << k):
                slot = ag_off[k] + j
                rdma = pltpu.make_async_remote_copy(
                    src_ref=o_ref.at[base + own_low + j * span],
                    dst_ref=o_ref.at[base + own_low + j * span],
                    send_sem=bag_send.at[slot],
                    recv_sem=bag_recv.at[slot],
                    device_id=(partner,),
                    device_id_type=pl.DeviceIdType.MESH,
                )
                rdma.start()
                rdmas.append(rdma)
            for jj, rdma in enumerate(rdmas):
                rdma.wait()
                c_desc(0, (1, 2, 4)[k] + jj).start()

        for h in (1, 2):
            for j in range(8):
                c_desc(h - 1, j).wait_recv()
                c_desc(h, j).start()
        for j in range(8):
            c_desc(2, j).wait_recv()
        for h in range(3):
            for j in range(8):
                c_desc(h, j).wait_send()

    return pl.pallas_call(
        body,
        out_shape=jax.ShapeDtypeStruct((n, rows, cols), dt),
        in_specs=[pl.BlockSpec(memory_space=pltpu.VMEM)],
        out_specs=pl.BlockSpec(memory_space=pltpu.VMEM),
        scratch_shapes=[
            pltpu.VMEM((3, 8, rows, cols), dt),
            pltpu.VMEM((3, 8, rows, cols), dt),
            pltpu.VMEM((7, rows, cols), dt),
            pltpu.SemaphoreType.DMA((3,)),
            pltpu.SemaphoreType.DMA((3,)),
            pltpu.SemaphoreType.DMA((7,)),
            pltpu.SemaphoreType.DMA((7,)),
            pltpu.SemaphoreType.DMA((7,)),
            pltpu.SemaphoreType.DMA((7,)),
            pltpu.SemaphoreType.DMA((24,)),
            pltpu.SemaphoreType.DMA((24,)),
        ],
        compiler_params=pltpu.CompilerParams(collective_id=0),
    )(partial)


def _ring_allreduce(partial):
    n, rows, cols = partial.shape
    dt = partial.dtype

    def body(p_ref, o_ref, comm_ref, rs_send, rs_recv, ag_send, ag_recv):
        my = lax.axis_index("i")
        left = (my + N_DEV - 1) % N_DEV
        right = (my + 1) % N_DEV

        barrier_sem = pltpu.get_barrier_semaphore()
        for nbr in (left, right):
            pl.semaphore_signal(
                barrier_sem, inc=1,
                device_id=(nbr,), device_id_type=pl.DeviceIdType.MESH,
            )
        pl.semaphore_wait(barrier_sem, 2)

        o_ref[...] = p_ref[...]

        for s in range(N_DEV - 1):
            sc = (my + 2 * N_DEV - s) % N_DEV
            rc = (my + 2 * N_DEV - s - 1) % N_DEV
            rdma = pltpu.make_async_remote_copy(
                src_ref=o_ref.at[sc],
                dst_ref=comm_ref.at[s],
                send_sem=rs_send.at[s],
                recv_sem=rs_recv.at[s],
                device_id=(right,),
                device_id_type=pl.DeviceIdType.MESH,
            )
            rdma.start()
            rdma.wait()
            o_ref[rc] = o_ref[rc] + comm_ref[s]

        for s in range(N_DEV - 1):
            sc = (my + 2 * N_DEV + 1 - s) % N_DEV
            rdma = pltpu.make_async_remote_copy(
                src_ref=o_ref.at[sc],
                dst_ref=o_ref.at[sc],
                send_sem=ag_send.at[s],
                recv_sem=ag_recv.at[s],
                device_id=(right,),
                device_id_type=pl.DeviceIdType.MESH,
            )
            rdma.start()
            rdma.wait()

    return pl.pallas_call(
        body,
        out_shape=jax.ShapeDtypeStruct((n, rows, cols), dt),
        in_specs=[pl.BlockSpec(memory_space=pltpu.VMEM)],
        out_specs=pl.BlockSpec(memory_space=pltpu.VMEM),
        scratch_shapes=[
            pltpu.VMEM((N_DEV - 1, rows, cols), dt),
            pltpu.SemaphoreType.DMA((N_DEV - 1,)),
            pltpu.SemaphoreType.DMA((N_DEV - 1,)),
            pltpu.SemaphoreType.DMA((N_DEV - 1,)),
            pltpu.SemaphoreType.DMA((N_DEV - 1,)),
        ],
        compiler_params=pltpu.CompilerParams(collective_id=0),
    )(partial)


def kernel(x, Wq, Wo, Wk, Wv):
    i = lax.axis_index("i")
    B, Sq, D = x.shape
    Hq_loc, Dh = 8, 128
    bf = jnp.bfloat16

    x2 = x.reshape(B * Sq, D).astype(bf)
    Q = (x2 @ Wq.astype(bf)).reshape(B, Sq, Hq_loc, Dh)

    Wk_s = lax.dynamic_slice_in_dim(Wk, i * 256, 256, axis=1).astype(bf)
    Wv_s = lax.dynamic_slice_in_dim(Wv, i * 256, 256, axis=1).astype(bf)
    K = (x2 @ Wk_s).reshape(B, Sq, 2, Dh)
    V = (x2 @ Wv_s).reshape(B, Sq, 2, Dh)
    Kr = jnp.repeat(K, 4, axis=2)
    Vr = jnp.repeat(V, 4, axis=2)

    s = jnp.einsum(
        "bihd,bjhd->bhij", Q, Kr, preferred_element_type=jnp.float32
    ) * SCALE
    p = jax.nn.softmax(s, axis=-1)
    o = jnp.einsum(
        "bhij,bjhd->bihd", p.astype(bf), Vr, preferred_element_type=jnp.float32
    )

    partial = jnp.dot(
        o.reshape(B * Sq, Hq_loc * Dh).astype(bf),
        Wo.astype(bf),
        preferred_element_type=jnp.float32,
    )

    red = _allreduce_2d(
        partial.reshape(N_DEV, (B * Sq) // N_DEV, D).astype(bf)
    )
    return red.astype(jnp.float32).reshape(B, Sq, D)


# device time: 88765 ns/iter; 2.6409x vs baseline; 1.0336x over previous
import jax
import jax.numpy as jnp
from jax import lax
from jax.experimental import pallas as pl
from jax.experimental.pallas import tpu as pltpu

N_DEV = 32
SCALE = 0.08838834764831843


def _allreduce_2d(partial):
    n, rows, cols = partial.shape
    dt = partial.dtype

    def body(p_ref, o_ref, comm_a, a_acc, comm_b,
             a_send, a_recv, brs_send, brs_recv, bag_send, bag_recv,
             c_send, c_recv):
        my = lax.axis_index("i")
        g = my % 8
        z = my // 8
        z_next = g + 8 * ((z + 1) % 4)
        z_prev = g + 8 * ((z + 3) % 4)

        barrier_sem = pltpu.get_barrier_semaphore()
        for nbr in (z_next, z_prev, 8 * z + (g ^ 1), 8 * z + (g ^ 2),
                    8 * z + (g ^ 4)):
            pl.semaphore_signal(
                barrier_sem, inc=1,
                device_id=(nbr,), device_id_type=pl.DeviceIdType.MESH,
            )
        pl.semaphore_wait(barrier_sem, 5)

        def a_desc(s, c):
            src = (p_ref.at[((z + 8 - s) % 4) * 8 + c] if s == 0
                   else a_acc.at[(s - 1) * 8 + c])
            return pltpu.make_async_remote_copy(
                src_ref=src,
                dst_ref=comm_a.at[s * 8 + c],
                send_sem=a_send.at[s * 8 + c],
                recv_sem=a_recv.at[s * 8 + c],
                device_id=(z_next,),
                device_id_type=pl.DeviceIdType.MESH,
            )

        for c in range(8):
            a_desc(0, c).start()
        for s in range(3):
            rt = (z + 8 - s - 1) % 4
            for c in range(8):
                a_desc(s, c).wait_recv()
                a_acc[s * 8 + c] = comm_a[s * 8 + c] + p_ref[rt * 8 + c]
                if s < 2:
                    a_desc(s + 1, c).start()
        for s in range(3):
            for c in range(8):
                a_desc(s, c).wait_send()

        t = (z + 1) % 4
        base = t * 8
        o_ref[pl.ds(base, 8)] = a_acc[pl.ds(16, 8)]

        rs_off = (0, 4, 6)
        for k in range(3):
            m = 1 << k
            width = 2 * m
            partner = 8 * z + (g ^ m)
            s_low = (g ^ m) % width
            r_low = g % width
            rdmas = []
            for j in range(8 // width):
                slot = rs_off[k] + j
                rdma = pltpu.make_async_remote_copy(
                    src_ref=o_ref.at[base + s_low + j * width],
                    dst_ref=comm_b.at[slot],
                    send_sem=brs_send.at[slot],
                    recv_sem=brs_recv.at[slot],
                    device_id=(partner,),
                    device_id_type=pl.DeviceIdType.MESH,
                )
                rdma.start()
                rdmas.append(rdma)
            for j, rdma in enumerate(rdmas):
                rdma.wait()
                rc = base + r_low + j * width
                o_ref[rc] = o_ref[rc] + comm_b[rs_off[k] + j]

        def slot_chunk(j):
            if j == 0:
                return g
            if j == 1:
                return g ^ 4
            if j < 4:
                return (g ^ 2) % 4 + 4 * (j - 2)
            return (g ^ 1) % 2 + 2 * (j - 4)

        def c_desc(h, j):
            idx = ((z + 9 - h) % 4) * 8 + slot_chunk(j)
            return pltpu.make_async_remote_copy(
                src_ref=o_ref.at[idx],
                dst_ref=o_ref.at[idx],
                send_sem=c_send.at[8 * h + j],
                recv_sem=c_recv.at[8 * h + j],
                device_id=(z_next,),
                device_id_type=pl.DeviceIdType.MESH,
            )

        c_desc(0, 0).start()

        ag_off = (0, 1, 3)
        for k in range(3):
            m = 4 >> k
            span = 8 >> k
            partner = 8 * z + (g ^ m)
            own_low = g % span
            rdmas = []
            for j in range(1 << k):
                slot = ag_off[k] + j
                rdma = pltpu.make_async_remote_copy(
                    src_ref=o_ref.at[base + own_low + j * span],
                    dst_ref=o_ref.at[base + own_low + j * span],
                    send_sem=bag_send.at[slot],
                    recv_sem=bag_recv.at[slot],
                    device_id=(partner,),
                    device_id_type=pl.DeviceIdType.MESH,
                )
                rdma.start()
                rdmas.append(rdma)
            for jj, rdma in enumerate(rdmas):
                rdma.wait()
                c_desc(0, (1, 2, 4)[k] + jj).start()

        for h in (1, 2):
            for j in range(8):
                c_desc(h - 1, j).wait_recv()
                c_desc(h, j).start()
        for j in range(8):
            c_desc(2, j).wait_recv()
        for h in range(3):
            for j in range(8):
                c_desc(h, j).wait_send()

    return pl.pallas_call(
        body,
        out_shape=jax.ShapeDtypeStruct((n, rows, cols), dt),
        in_specs=[pl.BlockSpec(memory_space=pltpu.VMEM)],
        out_specs=pl.BlockSpec(memory_space=pltpu.VMEM),
        scratch_shapes=[
            pltpu.VMEM((24, rows, cols), dt),
            pltpu.VMEM((24, rows, cols), dt),
            pltpu.VMEM((7, rows, cols), dt),
            pltpu.SemaphoreType.DMA((24,)),
            pltpu.SemaphoreType.DMA((24,)),
            pltpu.SemaphoreType.DMA((7,)),
            pltpu.SemaphoreType.DMA((7,)),
            pltpu.SemaphoreType.DMA((7,)),
            pltpu.SemaphoreType.DMA((7,)),
            pltpu.SemaphoreType.DMA((24,)),
            pltpu.SemaphoreType.DMA((24,)),
        ],
        compiler_params=pltpu.CompilerParams(collective_id=0),
    )(partial)


def _ring_allreduce(partial):
    n, rows, cols = partial.shape
    dt = partial.dtype

    def body(p_ref, o_ref, comm_ref, rs_send, rs_recv, ag_send, ag_recv):
        my = lax.axis_index("i")
        left = (my + N_DEV - 1) % N_DEV
        right = (my + 1) % N_DEV

        barrier_sem = pltpu.get_barrier_semaphore()
        for nbr in (left, right):
            pl.semaphore_signal(
                barrier_sem, inc=1,
                device_id=(nbr,), device_id_type=pl.DeviceIdType.MESH,
            )
        pl.semaphore_wait(barrier_sem, 2)

        o_ref[...] = p_ref[...]

        for s in range(N_DEV - 1):
            sc = (my + 2 * N_DEV - s) % N_DEV
            rc = (my + 2 * N_DEV - s - 1) % N_DEV
            rdma = pltpu.make_async_remote_copy(
                src_ref=o_ref.at[sc],
                dst_ref=comm_ref.at[s],
                send_sem=rs_send.at[s],
                recv_sem=rs_recv.at[s],
                device_id=(right,),
                device_id_type=pl.DeviceIdType.MESH,
            )
            rdma.start()
            rdma.wait()
            o_ref[rc] = o_ref[rc] + comm_ref[s]

        for s in range(N_DEV - 1):
            sc = (my + 2 * N_DEV + 1 - s) % N_DEV
            rdma = pltpu.make_async_remote_copy(
                src_ref=o_ref.at[sc],
                dst_ref=o_ref.at[sc],
                send_sem=ag_send.at[s],
                recv_sem=ag_recv.at[s],
                device_id=(right,),
                device_id_type=pl.DeviceIdType.MESH,
            )
            rdma.start()
            rdma.wait()

    return pl.pallas_call(
        body,
        out_shape=jax.ShapeDtypeStruct((n, rows, cols), dt),
        in_specs=[pl.BlockSpec(memory_space=pltpu.VMEM)],
        out_specs=pl.BlockSpec(memory_space=pltpu.VMEM),
        scratch_shapes=[
            pltpu.VMEM((N_DEV - 1, rows, cols), dt),
            pltpu.SemaphoreType.DMA((N_DEV - 1,)),
            pltpu.SemaphoreType.DMA((N_DEV - 1,)),
            pltpu.SemaphoreType.DMA((N_DEV - 1,)),
            pltpu.SemaphoreType.DMA((N_DEV - 1,)),
        ],
        compiler_params=pltpu.CompilerParams(collective_id=0),
    )(partial)


def kernel(x, Wq, Wo, Wk, Wv):
    i = lax.axis_index("i")
    B, Sq, D = x.shape
    Hq_loc, Dh = 8, 128
    bf = jnp.bfloat16

    x2 = x.reshape(B * Sq, D).astype(bf)
    Q = (x2 @ Wq.astype(bf)).reshape(B, Sq, Hq_loc, Dh)

    Wk_s = lax.dynamic_slice_in_dim(Wk, i * 256, 256, axis=1).astype(bf)
    Wv_s = lax.dynamic_slice_in_dim(Wv, i * 256, 256, axis=1).astype(bf)
    K = (x2 @ Wk_s).reshape(B, Sq, 2, Dh)
    V = (x2 @ Wv_s).reshape(B, Sq, 2, Dh)
    Kr = jnp.repeat(K, 4, axis=2)
    Vr = jnp.repeat(V, 4, axis=2)

    s = jnp.einsum(
        "bihd,bjhd->bhij", Q, Kr, preferred_element_type=jnp.float32
    ) * SCALE
    p = jax.nn.softmax(s, axis=-1)
    o = jnp.einsum(
        "bhij,bjhd->bihd", p.astype(bf), Vr, preferred_element_type=jnp.float32
    )

    partial = jnp.dot(
        o.reshape(B * Sq, Hq_loc * Dh).astype(bf),
        Wo.astype(bf),
        preferred_element_type=jnp.float32,
    )

    red = _allreduce_2d(
        partial.reshape(N_DEV, (B * Sq) // N_DEV, D).astype(bf)
    )
    return red.astype(jnp.float32).reshape(B, Sq, D)


# device time: 87247 ns/iter; 2.6868x vs baseline; 1.0174x over previous
import jax
import jax.numpy as jnp
from jax import lax
from jax.experimental import pallas as pl
from jax.experimental.pallas import tpu as pltpu

N_DEV = 32
SCALE = 0.08838834764831843


def _mm_allreduce_2d(o2, wo):
    rows, cols = 32, o2.shape[1]
    dt = o2.dtype

    def body(o2_ref, wo_ref, o_ref, p_loc, comm_a, a_acc, comm_b,
             a_send, a_recv, brs_send, brs_recv, bag_send, bag_recv,
             c_send, c_recv):
        my = lax.axis_index("i")
        g = my % 8
        z = my // 8
        z_next = g + 8 * ((z + 1) % 4)
        z_prev = g + 8 * ((z + 3) % 4)

        def mm(b):
            r = jnp.dot(
                o2_ref[pl.ds(b * 256, 256), :],
                wo_ref[...],
                preferred_element_type=jnp.float32,
            )
            p_loc[pl.ds(b * 8, 8)] = r.astype(dt).reshape(8, rows, cols)

        mm(z)

        barrier_sem = pltpu.get_barrier_semaphore()
        for nbr in (z_next, z_prev, 8 * z + (g ^ 1), 8 * z + (g ^ 2),
                    8 * z + (g ^ 4)):
            pl.semaphore_signal(
                barrier_sem, inc=1,
                device_id=(nbr,), device_id_type=pl.DeviceIdType.MESH,
            )
        pl.semaphore_wait(barrier_sem, 5)

        def a_desc(s, c):
            src = (p_loc.at[((z + 8 - s) % 4) * 8 + c] if s == 0
                   else a_acc.at[(s - 1) * 8 + c])
            return pltpu.make_async_remote_copy(
                src_ref=src,
                dst_ref=comm_a.at[s * 8 + c],
                send_sem=a_send.at[s * 8 + c],
                recv_sem=a_recv.at[s * 8 + c],
                device_id=(z_next,),
                device_id_type=pl.DeviceIdType.MESH,
            )

        for c in range(8):
            a_desc(0, c).start()
        for s in range(3):
            rt = (z + 8 - s - 1) % 4
            mm(rt)
            for c in range(8):
                a_desc(s, c).wait_recv()
                a_acc[s * 8 + c] = comm_a[s * 8 + c] + p_loc[rt * 8 + c]
                if s < 2:
                    a_desc(s + 1, c).start()
        for s in range(3):
            for c in range(8):
                a_desc(s, c).wait_send()

        t = (z + 1) % 4
        base = t * 8
        o_ref[pl.ds(base, 8)] = a_acc[pl.ds(16, 8)]

        rs_off = (0, 4, 6)
        for k in range(3):
            m = 1 << k
            width = 2 * m
            partner = 8 * z + (g ^ m)
            s_low = (g ^ m) % width
            r_low = g % width
            rdmas = []
            for j in range(8 // width):
                slot = rs_off[k] + j
                rdma = pltpu.make_async_remote_copy(
                    src_ref=o_ref.at[base + s_low + j * width],
                    dst_ref=comm_b.at[slot],
                    send_sem=brs_send.at[slot],
                    recv_sem=brs_recv.at[slot],
                    device_id=(partner,),
                    device_id_type=pl.DeviceIdType.MESH,
                )
                rdma.start()
                rdmas.append(rdma)
            for j, rdma in enumerate(rdmas):
                rdma.wait()
                rc = base + r_low + j * width
                o_ref[rc] = o_ref[rc] + comm_b[rs_off[k] + j]

        def slot_chunk(j):
            if j == 0:
                return g
            if j == 1:
                return g ^ 4
            if j < 4:
                return (g ^ 2) % 4 + 4 * (j - 2)
            return (g ^ 1) % 2 + 2 * (j - 4)

        def c_desc(h, j):
            idx = ((z + 9 - h) % 4) * 8 + slot_chunk(j)
            return pltpu.make_async_remote_copy(
                src_ref=o_ref.at[idx],
                dst_ref=o_ref.at[idx],
                send_sem=c_send.at[8 * h + j],
                recv_sem=c_recv.at[8 * h + j],
                device_id=(z_next,),
                device_id_type=pl.DeviceIdType.MESH,
            )

        c_desc(0, 0).start()

        ag_off = (0, 1, 3)
        for k in range(3):
            m = 4 >> k
            span = 8 >> k
            partner = 8 * z + (g ^ m)
            own_low = g % span
            rdmas = []
            for j in range(1 << k):
                slot = ag_off[k] + j
                rdma = pltpu.make_async_remote_copy(
                    src_ref=o_ref.at[base + own_low + j * span],
                    dst_ref=o_ref.at[base + own_low + j * span],
                    send_sem=bag_send.at[slot],
                    recv_sem=bag_recv.at[slot],
                    device_id=(partner,),
                    device_id_type=pl.DeviceIdType.MESH,
                )
                rdma.start()
                rdmas.append(rdma)
            for jj, rdma in enumerate(rdmas):
                rdma.wait()
                c_desc(0, (1, 2, 4)[k] + jj).start()

        for h in (1, 2):
            for j in range(8):
                c_desc(h - 1, j).wait_recv()
                c_desc(h, j).start()
        for j in range(8):
            c_desc(2, j).wait_recv()
        for h in range(3):
            for j in range(8):
                c_desc(h, j).wait_send()

    return pl.pallas_call(
        body,
        out_shape=jax.ShapeDtypeStruct((32, rows, cols), dt),
        in_specs=[
            pl.BlockSpec(memory_space=pltpu.VMEM),
            pl.BlockSpec(memory_space=pltpu.VMEM),
        ],
        out_specs=pl.BlockSpec(memory_space=pltpu.VMEM),
        scratch_shapes=[
            pltpu.VMEM((32, rows, cols), dt),
            pltpu.VMEM((24, rows, cols), dt),
            pltpu.VMEM((24, rows, cols), dt),
            pltpu.VMEM((7, rows, cols), dt),
            pltpu.SemaphoreType.DMA((24,)),
            pltpu.SemaphoreType.DMA((24,)),
            pltpu.SemaphoreType.DMA((7,)),
            pltpu.SemaphoreType.DMA((7,)),
            pltpu.SemaphoreType.DMA((7,)),
            pltpu.SemaphoreType.DMA((7,)),
            pltpu.SemaphoreType.DMA((24,)),
            pltpu.SemaphoreType.DMA((24,)),
        ],
        compiler_params=pltpu.CompilerParams(collective_id=0),
    )(o2, wo)


def _ring_allreduce(partial):
    n, rows, cols = partial.shape
    dt = partial.dtype

    def body(p_ref, o_ref, comm_ref, rs_send, rs_recv, ag_send, ag_recv):
        my = lax.axis_index("i")
        left = (my + N_DEV - 1) % N_DEV
        right = (my + 1) % N_DEV

        barrier_sem = pltpu.get_barrier_semaphore()
        for nbr in (left, right):
            pl.semaphore_signal(
                barrier_sem, inc=1,
                device_id=(nbr,), device_id_type=pl.DeviceIdType.MESH,
            )
        pl.semaphore_wait(barrier_sem, 2)

        o_ref[...] = p_ref[...]

        for s in range(N_DEV - 1):
            sc = (my + 2 * N_DEV - s) % N_DEV
            rc = (my + 2 * N_DEV - s - 1) % N_DEV
            rdma = pltpu.make_async_remote_copy(
                src_ref=o_ref.at[sc],
                dst_ref=comm_ref.at[s],
                send_sem=rs_send.at[s],
                recv_sem=rs_recv.at[s],
                device_id=(right,),
                device_id_type=pl.DeviceIdType.MESH,
            )
            rdma.start()
            rdma.wait()
            o_ref[rc] = o_ref[rc] + comm_ref[s]

        for s in range(N_DEV - 1):
            sc = (my + 2 * N_DEV + 1 - s) % N_DEV
            rdma = pltpu.make_async_remote_copy(
                src_ref=o_ref.at[sc],
                dst_ref=o_ref.at[sc],
                send_sem=ag_send.at[s],
                recv_sem=ag_recv.at[s],
                device_id=(right,),
                device_id_type=pl.DeviceIdType.MESH,
            )
            rdma.start()
            rdma.wait()

    return pl.pallas_call(
        body,
        out_shape=jax.ShapeDtypeStruct((n, rows, cols), dt),
        in_specs=[pl.BlockSpec(memory_space=pltpu.VMEM)],
        out_specs=pl.BlockSpec(memory_space=pltpu.VMEM),
        scratch_shapes=[
            pltpu.VMEM((N_DEV - 1, rows, cols), dt),
            pltpu.SemaphoreType.DMA((N_DEV - 1,)),
            pltpu.SemaphoreType.DMA((N_DEV - 1,)),
            pltpu.SemaphoreType.DMA((N_DEV - 1,)),
            pltpu.SemaphoreType.DMA((N_DEV - 1,)),
        ],
        compiler_params=pltpu.CompilerParams(collective_id=0),
    )(partial)


def kernel(x, Wq, Wo, Wk, Wv):
    i = lax.axis_index("i")
    B, Sq, D = x.shape
    Hq_loc, Dh = 8, 128
    bf = jnp.bfloat16

    x2 = x.reshape(B * Sq, D).astype(bf)
    Q = (x2 @ Wq.astype(bf)).reshape(B, Sq, Hq_loc, Dh)

    Wk_s = lax.dynamic_slice_in_dim(Wk, i * 256, 256, axis=1).astype(bf)
    Wv_s = lax.dynamic_slice_in_dim(Wv, i * 256, 256, axis=1).astype(bf)
    K = (x2 @ Wk_s).reshape(B, Sq, 2, Dh)
    V = (x2 @ Wv_s).reshape(B, Sq, 2, Dh)
    Kr = jnp.repeat(K, 4, axis=2)
    Vr = jnp.repeat(V, 4, axis=2)

    s = jnp.einsum(
        "bihd,bjhd->bhij", Q, Kr, preferred_element_type=jnp.float32
    ) * SCALE
    p = jax.nn.softmax(s, axis=-1)
    o = jnp.einsum(
        "bhij,bjhd->bihd", p.astype(bf), Vr, preferred_element_type=jnp.float32
    )

    red = _mm_allreduce_2d(
        o.reshape(B * Sq, Hq_loc * Dh).astype(bf), Wo.astype(bf)
    )
    return red.astype(jnp.float32).reshape(B, Sq, D)
